# Initial kernel scaffold; baseline (speedup 1.0000x reference)
import jax
import jax.numpy as jnp
from jax.experimental import pallas as pl


def kernel(feats, params, adj, edge_index, sorted_nodes, num_dst):
    raise NotImplementedError("write your pallas kernel here")


# speedup vs baseline: 2.9952x; 2.9952x over previous
import functools

import jax
import jax.numpy as jnp
from jax.experimental import pallas as pl
from jax.experimental.pallas import tpu as pltpu

N, D, H, CB, NUM_DST, OUT = 10000, 128, 256, 4096, 2048, 128


_BM = 256


def _vq_body(h_ref, wp_ref, bp_ref, cb_ref, wl_ref, bl_ref,
             logits_ref, quant_ref, hout_ref, avg_ref, commit_ref):
    i = pl.program_id(0)
    h = h_ref[...]
    logits = jnp.dot(h, wp_ref[...], preferred_element_type=jnp.float32) + bp_ref[...]
    logits_ref[...] = logits
    m = jnp.max(logits, axis=1, keepdims=True)
    e = jnp.exp(logits - m)
    s = jnp.sum(e, axis=1, keepdims=True)
    p = e / s
    quant_ref[...] = jnp.dot(p, cb_ref[...], preferred_element_type=jnp.float32)
    hout_ref[...] = jnp.dot(logits, wl_ref[...],
                            preferred_element_type=jnp.float32) + bl_ref[...]
    psum = jnp.sum(p, axis=0, keepdims=True)
    csum = jnp.sum(p * jnp.log(p * CB + 1e-10))

    @pl.when(i == 0)
    def _init():
        avg_ref[...] = psum
        commit_ref[...] = jnp.full((1, 1), csum, jnp.float32)

    @pl.when(i > 0)
    def _acc():
        avg_ref[...] += psum
        commit_ref[...] += jnp.full((1, 1), csum, jnp.float32)


def _vq_stage(h, wp, bp, cb, wl, bl):
    M = h.shape[0]
    grid = M // _BM
    out_shapes = (
        jax.ShapeDtypeStruct((M, CB), jnp.float32),
        jax.ShapeDtypeStruct((M, H), jnp.float32),
        jax.ShapeDtypeStruct((M, OUT), jnp.float32),
        jax.ShapeDtypeStruct((1, CB), jnp.float32),
        jax.ShapeDtypeStruct((1, 1), jnp.float32),
    )
    return pl.pallas_call(
        _vq_body,
        grid=(grid,),
        in_specs=[
            pl.BlockSpec((_BM, H), lambda i: (i, 0)),
            pl.BlockSpec((H, CB), lambda i: (0, 0)),
            pl.BlockSpec((1, CB), lambda i: (0, 0)),
            pl.BlockSpec((CB, H), lambda i: (0, 0)),
            pl.BlockSpec((CB, OUT), lambda i: (0, 0)),
            pl.BlockSpec((1, OUT), lambda i: (0, 0)),
        ],
        out_specs=(
            pl.BlockSpec((_BM, CB), lambda i: (i, 0)),
            pl.BlockSpec((_BM, H), lambda i: (i, 0)),
            pl.BlockSpec((_BM, OUT), lambda i: (i, 0)),
            pl.BlockSpec((1, CB), lambda i: (0, 0)),
            pl.BlockSpec((1, 1), lambda i: (0, 0)),
        ),
        out_shape=out_shapes,
    )(h, wp, bp.reshape(1, CB), cb, wl, bl.reshape(1, OUT))


def _aggregate(h, src, dst, c, norm):
    msg = h * norm[:, None]
    agg = c[:, None] * msg
    agg = agg.at[dst].add(msg[src])
    agg = agg.at[src].add(msg[dst])
    return agg * norm[:, None]


def _bn(x, g, b, eps=1e-5):
    m = jnp.mean(x, 0)
    v = jnp.var(x, 0)
    return (x - m) / jnp.sqrt(v + eps) * g + b


def _l2n(x):
    return x / jnp.maximum(jnp.linalg.norm(x, axis=-1, keepdims=True), 1e-12)


def kernel(feats, params, adj, edge_index, sorted_nodes, num_dst):
    p = params
    src = edge_index[0]
    dst = edge_index[1]
    ones = jnp.ones(src.shape[0], jnp.float32)
    c = jnp.zeros(N, jnp.float32).at[src].add(ones).at[dst].add(ones)
    norm = jnp.where(c > 0, (2.0 * c) ** -0.5, 1.0)

    aggA = _aggregate(feats, src, dst, c, norm)
    h_node = _bn(jax.nn.relu(aggA @ p['W1'] + p['b1']), p['g1'], p['be1'])
    h_link = _bn(jax.nn.relu(aggA @ p['W3'] + p['b3']), p['g3'], p['be3'])

    aggB = _aggregate(h_node, src, dst, c, norm)
    h_node = _bn(jax.nn.relu(aggB @ p['W2'] + p['b2']), p['g2'], p['be2'])
    aggC = _aggregate(h_link, src, dst, c, norm)
    h_link = _bn(jax.nn.relu(aggC @ p['W4'] + p['b4']), p['g4'], p['be4'])

    start = num_dst - NUM_DST
    h = jnp.concatenate([
        jax.lax.dynamic_slice_in_dim(h_node, start, NUM_DST, 0),
        jax.lax.dynamic_slice_in_dim(h_link, start, NUM_DST, 0)], 0)

    logits, quantized, hout_full, avg_sum, commit_sum = _vq_stage(
        h, p['Wp'], p['bp'], p['codebook'], p['Wl'], p['bl'])

    Mrows = 2 * NUM_DST
    avg = avg_sum[0] / Mrows
    perplexity = jnp.exp(-jnp.sum(avg * jnp.log(avg + 1e-10)))
    commit_loss = 0.25 * commit_sum[0, 0] / Mrows

    quantized_node = (
        jax.lax.dynamic_slice_in_dim(quantized, start, NUM_DST, 0) @ p['Wd1']
        + p['bd1'])
    cos = jnp.sum(
        _l2n(jax.lax.dynamic_slice_in_dim(feats, start, NUM_DST, 0))
        * _l2n(quantized_node), 1)
    feature_rec_loss = 10.0 * jnp.mean((1.0 - cos) ** 2)

    quantized_edge = (
        jax.lax.dynamic_slice_in_dim(quantized, num_dst, NUM_DST, 0) @ p['Wd2']
        + p['bd2'])[sorted_nodes]
    adj_q = quantized_edge @ quantized_edge.T
    adj_q = (adj_q - adj_q.min()) / (adj_q.max() - adj_q.min())
    edge_rec_loss = jnp.sqrt(jnp.mean((adj - adj_q) ** 2))

    h_out = jax.lax.dynamic_slice_in_dim(hout_full, start, NUM_DST, 0)
    loss = feature_rec_loss + edge_rec_loss + commit_loss
    return h_out, loss, logits, p['codebook'], perplexity, quantized


# speedup vs baseline: 5.4504x; 1.8197x over previous
import functools

import jax
import jax.numpy as jnp
from jax import lax
from jax.experimental import pallas as pl
from jax.experimental.pallas import tpu as pltpu
from jax.experimental.pallas import tpu_sc as plsc

N, D, H, CB, NUM_DST, OUT = 10000, 128, 256, 4096, 2048, 128


_NC, _NT = 2, 16
_NPAD = 10112
_RPT = _NPAD // _NT
_K = 128
_NCHUNK = 160
_EPAD = _NT * _NCHUNK * _K


_ICH = 40


def _agg_sc_body(nck, edge_mode, msgL, msgR, srcm, dstm, zeros_hbm, outL, outR,
                 sidx, didx, rows_s, rows_d, acc, sem1, sem2):
    cid = lax.axis_index("c")
    sid = lax.axis_index("s")
    r0 = sid * _RPT
    t0 = sid * _NCHUNK
    if edge_mode:
        t0 = t0 + cid * nck

    def run(msg_hbm, out_hbm):
        pltpu.sync_copy(zeros_hbm.at[pl.ds(r0, _RPT)], acc.at[pl.ds(r0, _RPT)])
        plsc.subcore_barrier()

        def group(g, carry):
            g0 = t0 + g * _ICH
            pltpu.sync_copy(srcm.at[pl.ds(g0, _ICH)], sidx)
            pltpu.sync_copy(dstm.at[pl.ds(g0, _ICH)], didx)

            def chunk(k, carry2):
                pltpu.async_copy(msg_hbm.at[sidx.at[k]], rows_s, sem1).wait()
                pltpu.async_copy(msg_hbm.at[didx.at[k]], rows_d, sem2).wait()
                pltpu.sync_copy(rows_s, acc.at[didx.at[k]], add=True)
                pltpu.sync_copy(rows_d, acc.at[sidx.at[k]], add=True)
                return carry2

            lax.fori_loop(0, _ICH, chunk, 0)
            return carry

        lax.fori_loop(0, nck // _ICH, group, 0)
        plsc.subcore_barrier()
        pltpu.sync_copy(acc.at[pl.ds(r0, _RPT)], out_hbm.at[pl.ds(r0, _RPT)])

    @pl.when(cid == 0)
    def _c0():
        run(msgL, outL)

    @pl.when(cid == 1)
    def _c1():
        run(msgR, outR)


def _make_agg_sc(dh, edge_mode):
    nck = _NCHUNK // 2 if edge_mode else _NCHUNK
    mesh = plsc.VectorSubcoreMesh(core_axis_name="c", subcore_axis_name="s")
    return pl.kernel(
        functools.partial(_agg_sc_body, nck, edge_mode),
        out_type=(jax.ShapeDtypeStruct((_NPAD, dh), jnp.float32),
                  jax.ShapeDtypeStruct((_NPAD, dh), jnp.float32)),
        mesh=mesh,
        scratch_types=[
            pltpu.VMEM((_ICH, _K), jnp.int32),
            pltpu.VMEM((_ICH, _K), jnp.int32),
            pltpu.VMEM((_K, dh), jnp.float32),
            pltpu.VMEM((_K, dh), jnp.float32),
            pltpu.VMEM_SHARED((_NPAD, dh), jnp.float32),
            pltpu.SemaphoreType.DMA,
            pltpu.SemaphoreType.DMA,
        ],
    )


def _sc_aggregate(h, srcm, dstm, c, norm):
    dm = h.shape[1]
    msg = h * norm[:, None]
    pad = ((0, _NPAD - N), (0, 0))
    if dm >= 256:
        dh = dm // 2
        msgL = jnp.pad(msg[:, :dh], pad)
        msgR = jnp.pad(msg[:, dh:], pad)
        zeros = jnp.zeros((_NPAD, dh), jnp.float32)
        outL, outR = _make_agg_sc(dh, False)(msgL, msgR, srcm, dstm, zeros)
        neigh = jnp.concatenate([outL[:N], outR[:N]], 1)
    else:
        msgp = jnp.pad(msg, pad)
        zeros = jnp.zeros((_NPAD, dm), jnp.float32)
        outL, outR = _make_agg_sc(dm, True)(msgp, msgp, srcm, dstm, zeros)
        neigh = (outL + outR)[:N]
    return (c[:, None] * msg + neigh) * norm[:, None]


_BM = 256


def _vq_body(h_ref, wp_ref, bp_ref, cb_ref, wl_ref, bl_ref,
             logits_ref, quant_ref, hout_ref, avg_ref, commit_ref):
    i = pl.program_id(0)
    h = h_ref[...]
    logits = jnp.dot(h, wp_ref[...], preferred_element_type=jnp.float32) + bp_ref[...]
    logits_ref[...] = logits
    m = jnp.max(logits, axis=1, keepdims=True)
    e = jnp.exp(logits - m)
    s = jnp.sum(e, axis=1, keepdims=True)
    p = e / s
    quant_ref[...] = jnp.dot(p, cb_ref[...], preferred_element_type=jnp.float32)
    hout_ref[...] = jnp.dot(logits, wl_ref[...],
                            preferred_element_type=jnp.float32) + bl_ref[...]
    psum = jnp.sum(p, axis=0, keepdims=True)
    csum = jnp.sum(p * jnp.log(p * CB + 1e-10))

    @pl.when(i == 0)
    def _init():
        avg_ref[...] = psum
        commit_ref[...] = jnp.full((1, 1), csum, jnp.float32)

    @pl.when(i > 0)
    def _acc():
        avg_ref[...] += psum
        commit_ref[...] += jnp.full((1, 1), csum, jnp.float32)


def _vq_stage(h, wp, bp, cb, wl, bl):
    M = h.shape[0]
    grid = M // _BM
    out_shapes = (
        jax.ShapeDtypeStruct((M, CB), jnp.float32),
        jax.ShapeDtypeStruct((M, H), jnp.float32),
        jax.ShapeDtypeStruct((M, OUT), jnp.float32),
        jax.ShapeDtypeStruct((1, CB), jnp.float32),
        jax.ShapeDtypeStruct((1, 1), jnp.float32),
    )
    return pl.pallas_call(
        _vq_body,
        grid=(grid,),
        in_specs=[
            pl.BlockSpec((_BM, H), lambda i: (i, 0)),
            pl.BlockSpec((H, CB), lambda i: (0, 0)),
            pl.BlockSpec((1, CB), lambda i: (0, 0)),
            pl.BlockSpec((CB, H), lambda i: (0, 0)),
            pl.BlockSpec((CB, OUT), lambda i: (0, 0)),
            pl.BlockSpec((1, OUT), lambda i: (0, 0)),
        ],
        out_specs=(
            pl.BlockSpec((_BM, CB), lambda i: (i, 0)),
            pl.BlockSpec((_BM, H), lambda i: (i, 0)),
            pl.BlockSpec((_BM, OUT), lambda i: (i, 0)),
            pl.BlockSpec((1, CB), lambda i: (0, 0)),
            pl.BlockSpec((1, 1), lambda i: (0, 0)),
        ),
        out_shape=out_shapes,
    )(h, wp, bp.reshape(1, CB), cb, wl, bl.reshape(1, OUT))


def _aggregate(h, src, dst, c, norm):
    msg = h * norm[:, None]
    agg = c[:, None] * msg
    agg = agg.at[dst].add(msg[src])
    agg = agg.at[src].add(msg[dst])
    return agg * norm[:, None]


def _bn(x, g, b, eps=1e-5):
    m = jnp.mean(x, 0)
    v = jnp.var(x, 0)
    return (x - m) / jnp.sqrt(v + eps) * g + b


def _l2n(x):
    return x / jnp.maximum(jnp.linalg.norm(x, axis=-1, keepdims=True), 1e-12)


def kernel(feats, params, adj, edge_index, sorted_nodes, num_dst):
    p = params
    src = edge_index[0]
    dst = edge_index[1]
    E = src.shape[0]
    ones = jnp.ones(E, jnp.float32)
    c = jnp.zeros(N, jnp.float32).at[src].add(ones).at[dst].add(ones)
    norm = jnp.where(c > 0, (2.0 * c) ** -0.5, 1.0)

    srcm = jnp.full((_EPAD,), N, jnp.int32).at[:E].set(src).reshape(
        _NT * _NCHUNK, _K)
    dstm = jnp.full((_EPAD,), N, jnp.int32).at[:E].set(dst).reshape(
        _NT * _NCHUNK, _K)

    aggA = _sc_aggregate(feats, srcm, dstm, c, norm)
    h_node = _bn(jax.nn.relu(aggA @ p['W1'] + p['b1']), p['g1'], p['be1'])
    h_link = _bn(jax.nn.relu(aggA @ p['W3'] + p['b3']), p['g3'], p['be3'])

    aggB = _sc_aggregate(h_node, srcm, dstm, c, norm)
    h_node = _bn(jax.nn.relu(aggB @ p['W2'] + p['b2']), p['g2'], p['be2'])
    aggC = _sc_aggregate(h_link, srcm, dstm, c, norm)
    h_link = _bn(jax.nn.relu(aggC @ p['W4'] + p['b4']), p['g4'], p['be4'])

    start = num_dst - NUM_DST
    h = jnp.concatenate([
        jax.lax.dynamic_slice_in_dim(h_node, start, NUM_DST, 0),
        jax.lax.dynamic_slice_in_dim(h_link, start, NUM_DST, 0)], 0)

    logits, quantized, hout_full, avg_sum, commit_sum = _vq_stage(
        h, p['Wp'], p['bp'], p['codebook'], p['Wl'], p['bl'])

    Mrows = 2 * NUM_DST
    avg = avg_sum[0] / Mrows
    perplexity = jnp.exp(-jnp.sum(avg * jnp.log(avg + 1e-10)))
    commit_loss = 0.25 * commit_sum[0, 0] / Mrows

    quantized_node = (
        jax.lax.dynamic_slice_in_dim(quantized, start, NUM_DST, 0) @ p['Wd1']
        + p['bd1'])
    cos = jnp.sum(
        _l2n(jax.lax.dynamic_slice_in_dim(feats, start, NUM_DST, 0))
        * _l2n(quantized_node), 1)
    feature_rec_loss = 10.0 * jnp.mean((1.0 - cos) ** 2)

    quantized_edge = (
        jax.lax.dynamic_slice_in_dim(quantized, num_dst, NUM_DST, 0) @ p['Wd2']
        + p['bd2'])[sorted_nodes]
    adj_q = quantized_edge @ quantized_edge.T
    adj_q = (adj_q - adj_q.min()) / (adj_q.max() - adj_q.min())
    edge_rec_loss = jnp.sqrt(jnp.mean((adj - adj_q) ** 2))

    h_out = jax.lax.dynamic_slice_in_dim(hout_full, start, NUM_DST, 0)
    loss = feature_rec_loss + edge_rec_loss + commit_loss
    return h_out, loss, logits, p['codebook'], perplexity, quantized
